# Initial kernel scaffold; baseline (speedup 1.0000x reference)
import jax
import jax.numpy as jnp
from jax.experimental import pallas as pl


def kernel(events):
    raise NotImplementedError("write your pallas kernel here")


# speedup vs baseline: 4.4682x; 4.4682x over previous
import dataclasses
import functools

import jax
import jax.numpy as jnp
from jax import lax
from jax.experimental import pallas as pl
from jax.experimental.pallas import tpu as pltpu
from jax.experimental.pallas import tpu_sc as plsc

NLANE = 16
NSUB = 16
NCORE = 2


def _prep_call(G, SBR, W, HW, SURF, interpret=False):
  def body(ev_ref, idx_ref, t_ref):
    a = ev_ref[...]
    xi = a[0].astype(jnp.int32)
    yi = a[1].astype(jnp.int32)
    pi = a[3].astype(jnp.int32)
    bi = a[4].astype(jnp.int32)
    idx_ref[...] = bi * SURF + pi * HW + yi * W + xi
    t_ref[...] = a[2]

  return pl.pallas_call(
      body,
      grid=(G,),
      in_specs=[pl.BlockSpec((5, 1, SBR, 128), lambda i: (0, i, 0, 0))],
      out_specs=[
          pl.BlockSpec((1, SBR, 128), lambda i: (i, 0, 0)),
          pl.BlockSpec((1, SBR, 128), lambda i: (i, 0, 0)),
      ],
      out_shape=[
          jax.ShapeDtypeStruct((G, SBR, 128), jnp.int32),
          jax.ShapeDtypeStruct((G, SBR, 128), jnp.float32),
      ],
      interpret=interpret,
  )


def _sc_call(NP, Bn, SURF, E, interpret=False):
  CPT = SURF // NSUB
  BPC = Bn // NCORE
  NBND = 16 * Bn * NSUB // NSUB
  del NBND
  BIG = jnp.float32(3.0e38)
  mesh = plsc.VectorSubcoreMesh(core_axis_name="core",
                                subcore_axis_name="subcore",
                                num_cores=NCORE, num_subcores=NSUB)
  cp = pltpu.CompilerParams()
  if "needs_layout_passes" in pltpu.CompilerParams.__dataclass_fields__:
    cp = dataclasses.replace(cp, needs_layout_passes=False)

  @functools.partial(
      pl.kernel,
      out_type=jax.ShapeDtypeStruct((Bn * SURF,), jnp.float32),
      mesh=mesh,
      compiler_params=cp,
      scratch_types=[
          pltpu.VMEM((544,), jnp.int32),
          pltpu.VMEM((E + 16,), jnp.int32),
          pltpu.VMEM((E + 16,), jnp.float32),
          pltpu.VMEM((CPT,), jnp.float32),
          pltpu.VMEM((NLANE,), jnp.float32),
          pltpu.VMEM((NSUB * NLANE,), jnp.float32),
          pltpu.VMEM_SHARED((NSUB * NLANE,), jnp.float32),
      ],
      interpret=interpret,
  )
  def call(si_hbm, st_hbm, bounds_hbm, out_hbm,
           boundsv, cellb, tb, slab, stage, minall, minshared):
    c = lax.axis_index("core")
    s = lax.axis_index("subcore")
    pltpu.sync_copy(bounds_hbm, boundsv)
    lanes = lax.iota(jnp.int32, NLANE)

    @pl.loop(0, BPC)
    def _batch(j):
      bi = c * BPC + j
      k = bi * NSUB + s
      kv = jnp.full((NLANE,), k, jnp.int32)
      bstart = jnp.min(plsc.load_gather(boundsv, [kv]), axis=0)
      bend = jnp.min(plsc.load_gather(boundsv, [kv + 1]), axis=0)
      lo = k * CPT

      @pl.loop(0, CPT, step=NLANE)
      def _zero(i):
        slab[pl.ds(i, NLANE)] = jnp.zeros((NLANE,), jnp.float32)

      start8 = bstart & jnp.int32(-8)
      nch = (bend - start8 + (E - 1)) // E

      def _chunk(k2, macc):
        off = pl.multiple_of(start8 + k2 * E, 8)
        pltpu.sync_copy(si_hbm.at[pl.ds(off, E + 16)], cellb)
        pltpu.sync_copy(st_hbm.at[pl.ds(off, E + 16)], tb)

        def _vec(v, macc):
          cv = cellb[pl.ds(v * NLANE, NLANE)]
          tv = tb[pl.ds(v * NLANE, NLANE)]
          nxt = plsc.load_gather(cellb, [lanes + (v * NLANE + 1)])
          pos = off + v * NLANE + lanes
          keep = (pos >= bstart) & (pos < bend) & (cv != nxt)
          loc = jnp.where(keep, cv - lo, 0)
          plsc.store_scatter(slab, [loc], tv, mask=keep)
          return jnp.minimum(macc, jnp.where(keep, tv, BIG))

        return lax.fori_loop(0, E // NLANE, _vec, macc)

      macc = lax.fori_loop(0, nch, _chunk,
                           jnp.full((NLANE,), BIG, jnp.float32))
      lmin = jnp.min(macc, axis=0)
      stage[...] = jnp.full((NLANE,), lmin, jnp.float32)
      pltpu.sync_copy(stage,
                      minshared.at[pl.ds(pl.multiple_of(s * NLANE, 8),
                                         NLANE)])
      plsc.subcore_barrier()
      pltpu.sync_copy(minshared, minall)
      plsc.subcore_barrier()

      def _min2(i, acc):
        return jnp.minimum(acc, minall[pl.ds(i * NLANE, NLANE)])
      acc2 = lax.fori_loop(0, NSUB, _min2,
                           jnp.full((NLANE,), BIG, jnp.float32))
      bmin = jnp.min(acc2, axis=0)

      @pl.loop(0, CPT, step=NLANE)
      def _sub(i):
        vv = slab[pl.ds(i, NLANE)]
        slab[pl.ds(i, NLANE)] = jnp.maximum(vv - bmin, jnp.float32(0.0))

      pltpu.sync_copy(
          slab, out_hbm.at[pl.ds(pl.multiple_of(bi * SURF + s * CPT, 8),
                                 CPT)])

  return call


def _run(events, Bn, H, W, SBR, E, interpret=False):
  N, five = events.shape
  assert five == 5 and N % 128 == 0
  R = N // 128
  assert R % SBR == 0
  G = R // SBR
  HW = H * W
  SURF = 2 * HW
  CPT = SURF // NSUB
  evT4 = events.T.reshape(5, G, SBR, 128)
  idx4, t4 = _prep_call(G, SBR, W, HW, SURF, interpret=interpret)(evT4)
  si, st = lax.sort([idx4.reshape(N), t4.reshape(N)],
                    dimension=0, num_keys=1, is_stable=False)
  nseg = Bn * NSUB
  bounds = jnp.searchsorted(
      si, jnp.arange(nseg + 1, dtype=jnp.int32) * CPT,
      side="left").astype(jnp.int32)
  bounds = jnp.concatenate(
      [bounds, jnp.full((544 - (nseg + 1),), N, jnp.int32)])
  PAD = E + 32
  si_p = jnp.concatenate(
      [si, jnp.full((PAD,), jnp.int32(2**31 - 1))])
  st_p = jnp.concatenate([st, jnp.zeros((PAD,), jnp.float32)])
  out = _sc_call(N + PAD, Bn, SURF, E, interpret=interpret)(
      si_p, st_p, bounds)
  return out.reshape(Bn, 2, H, W)


def kernel(events):
  return _run(events, Bn=32, H=480, W=640, SBR=125, E=8192)


# speedup vs baseline: 4.7927x; 1.0726x over previous
import dataclasses
import functools

import jax
import jax.numpy as jnp
from jax import lax
from jax.experimental import pallas as pl
from jax.experimental.pallas import tpu as pltpu
from jax.experimental.pallas import tpu_sc as plsc

NLANE = 16
NSUB = 16
NCORE = 2


def _prep_call(G, SBR, W, HW, SURF, interpret=False):
  def body(ev_ref, idx_ref, t_ref):
    a = ev_ref[...]
    xi = a[0].astype(jnp.int32)
    yi = a[1].astype(jnp.int32)
    pi = a[3].astype(jnp.int32)
    bi = a[4].astype(jnp.int32)
    idx_ref[...] = bi * SURF + pi * HW + yi * W + xi
    t_ref[...] = a[2]

  return pl.pallas_call(
      body,
      grid=(G,),
      in_specs=[pl.BlockSpec((5, 1, SBR, 128), lambda i: (0, i, 0, 0))],
      out_specs=[
          pl.BlockSpec((1, SBR, 128), lambda i: (i, 0, 0)),
          pl.BlockSpec((1, SBR, 128), lambda i: (i, 0, 0)),
      ],
      out_shape=[
          jax.ShapeDtypeStruct((G, SBR, 128), jnp.int32),
          jax.ShapeDtypeStruct((G, SBR, 128), jnp.float32),
      ],
      interpret=interpret,
  )


def _sc_call(NP, Bn, SURF, E, interpret=False):
  CPT = SURF // NSUB
  BPC = Bn // NCORE
  NBND = 16 * Bn * NSUB // NSUB
  del NBND
  BIG = jnp.float32(3.0e38)
  mesh = plsc.VectorSubcoreMesh(core_axis_name="core",
                                subcore_axis_name="subcore",
                                num_cores=NCORE, num_subcores=NSUB)
  cp = pltpu.CompilerParams()
  if "needs_layout_passes" in pltpu.CompilerParams.__dataclass_fields__:
    cp = dataclasses.replace(cp, needs_layout_passes=False)

  @functools.partial(
      pl.kernel,
      out_type=jax.ShapeDtypeStruct((Bn * SURF,), jnp.float32),
      mesh=mesh,
      compiler_params=cp,
      scratch_types=[
          pltpu.VMEM((544,), jnp.int32),
          pltpu.VMEM((E + 16,), jnp.int32),
          pltpu.VMEM((E + 16,), jnp.float32),
          pltpu.VMEM((CPT,), jnp.float32),
          pltpu.VMEM((CPT,), jnp.float32),
          pltpu.VMEM((NLANE,), jnp.float32),
          pltpu.VMEM((NSUB * NLANE,), jnp.float32),
          pltpu.VMEM_SHARED((NSUB * NLANE,), jnp.float32),
          pltpu.SemaphoreType.DMA,
          pltpu.SemaphoreType.DMA,
          pltpu.SemaphoreType.DMA,
      ],
      interpret=interpret,
  )
  def call(si_hbm, st_hbm, bounds_hbm, out_hbm,
           boundsv, cellb, tb, slab, outb, stage, minall, minshared,
           sem_i, sem_t, sem_o):
    c = lax.axis_index("core")
    s = lax.axis_index("subcore")
    pltpu.sync_copy(bounds_hbm, boundsv)
    lanes = lax.iota(jnp.int32, NLANE)

    @pl.loop(0, CPT, step=NLANE, unroll=8)
    def _zero0(i):
      slab[pl.ds(i, NLANE)] = jnp.zeros((NLANE,), jnp.float32)

    def _out_dma(bi):
      dst = out_hbm.at[pl.ds(pl.multiple_of(bi * SURF + s * CPT, 8), CPT)]
      return pltpu.make_async_copy(outb, dst, sem_o)

    @pl.loop(0, BPC)
    def _batch(j):
      bi = c * BPC + j
      k = bi * NSUB + s
      kv = jnp.full((NLANE,), k, jnp.int32)
      bstart = jnp.min(plsc.load_gather(boundsv, [kv]), axis=0)
      bend = jnp.min(plsc.load_gather(boundsv, [kv + 1]), axis=0)
      lo = k * CPT

      start8 = bstart & jnp.int32(-8)
      nch = (bend - start8 + (E - 1)) // E

      def _chunk(k2, macc):
        off = pl.multiple_of(start8 + k2 * E, 8)
        ca = pltpu.make_async_copy(si_hbm.at[pl.ds(off, E + 16)],
                                   cellb, sem_i)
        cb = pltpu.make_async_copy(st_hbm.at[pl.ds(off, E + 16)],
                                   tb, sem_t)
        ca.start()
        cb.start()
        ca.wait()
        cb.wait()

        def _vec(v, macc):
          cv = cellb[pl.ds(v, NLANE)]
          tv = tb[pl.ds(v, NLANE)]
          nxt = plsc.load_gather(cellb, [lanes + (v + 1)])
          pos = off + v + lanes
          keep = (pos >= bstart) & (pos < bend) & (cv != nxt)
          loc = jnp.where(keep, cv - lo, 0)
          plsc.store_scatter(slab, [loc], tv, mask=keep)
          return jnp.minimum(macc, jnp.where(keep, tv, BIG))

        return pl.loop(0, E, step=NLANE, init_carry=macc,
                       unroll=2)(_vec)

      macc = lax.fori_loop(0, nch, _chunk,
                           jnp.full((NLANE,), BIG, jnp.float32))
      lmin = jnp.min(macc, axis=0)
      stage[...] = jnp.full((NLANE,), lmin, jnp.float32)
      pltpu.sync_copy(stage,
                      minshared.at[pl.ds(pl.multiple_of(s * NLANE, 8),
                                         NLANE)])
      plsc.subcore_barrier()
      pltpu.sync_copy(minshared, minall)
      plsc.subcore_barrier()

      def _min2(i, acc):
        return jnp.minimum(acc, minall[pl.ds(i * NLANE, NLANE)])
      acc2 = lax.fori_loop(0, NSUB, _min2,
                           jnp.full((NLANE,), BIG, jnp.float32))
      bmin = jnp.min(acc2, axis=0)

      @pl.when(j > 0)
      def _():
        _out_dma(bi - 1).wait()

      @pl.loop(0, CPT, step=NLANE, unroll=8)
      def _sub(i):
        vv = slab[pl.ds(i, NLANE)]
        outb[pl.ds(i, NLANE)] = jnp.maximum(vv - bmin, jnp.float32(0.0))
        slab[pl.ds(i, NLANE)] = jnp.zeros((NLANE,), jnp.float32)

      _out_dma(bi).start()

    _out_dma(c * BPC + BPC - 1).wait()

  return call


def _run(events, Bn, H, W, SBR, E, interpret=False):
  N, five = events.shape
  assert five == 5 and N % 128 == 0
  R = N // 128
  assert R % SBR == 0
  G = R // SBR
  HW = H * W
  SURF = 2 * HW
  CPT = SURF // NSUB
  evT4 = events.T.reshape(5, G, SBR, 128)
  idx4, t4 = _prep_call(G, SBR, W, HW, SURF, interpret=interpret)(evT4)
  si, st = lax.sort([idx4.reshape(N), t4.reshape(N)],
                    dimension=0, num_keys=1, is_stable=False)
  nseg = Bn * NSUB
  bounds = jnp.searchsorted(
      si, jnp.arange(nseg + 1, dtype=jnp.int32) * CPT,
      side="left").astype(jnp.int32)
  bounds = jnp.concatenate(
      [bounds, jnp.full((544 - (nseg + 1),), N, jnp.int32)])
  PAD = E + 32
  si_p = jnp.concatenate(
      [si, jnp.full((PAD,), jnp.int32(2**31 - 1))])
  st_p = jnp.concatenate([st, jnp.zeros((PAD,), jnp.float32)])
  out = _sc_call(N + PAD, Bn, SURF, E, interpret=interpret)(
      si_p, st_p, bounds)
  return out.reshape(Bn, 2, H, W)


def kernel(events):
  return _run(events, Bn=32, H=480, W=640, SBR=125, E=8192)


# speedup vs baseline: 5.0177x; 1.0469x over previous
import dataclasses
import functools

import jax
import jax.numpy as jnp
from jax import lax
from jax.experimental import pallas as pl
from jax.experimental.pallas import tpu as pltpu
from jax.experimental.pallas import tpu_sc as plsc

NLANE = 16
NSUB = 16
NCORE = 2


def _prep_call(N, BL, W, HW, SURF, interpret=False):
  G = (N + BL - 1) // BL

  def body(ev_ref, idx_ref, t_ref):
    a = ev_ref[...]
    xi = a[0].astype(jnp.int32)
    yi = a[1].astype(jnp.int32)
    pi = a[3].astype(jnp.int32)
    bi = a[4].astype(jnp.int32)
    idx_ref[...] = bi * SURF + pi * HW + yi * W + xi
    t_ref[...] = a[2]

  return pl.pallas_call(
      body,
      grid=(G,),
      in_specs=[pl.BlockSpec((5, BL), lambda i: (0, i))],
      out_specs=[
          pl.BlockSpec((BL,), lambda i: (i,)),
          pl.BlockSpec((BL,), lambda i: (i,)),
      ],
      out_shape=[
          jax.ShapeDtypeStruct((N,), jnp.int32),
          jax.ShapeDtypeStruct((N,), jnp.float32),
      ],
      interpret=interpret,
  )


def _sc_call(NP, Bn, SURF, E, interpret=False):
  CPT = SURF // NSUB
  BPC = Bn // NCORE
  NBND = 16 * Bn * NSUB // NSUB
  del NBND
  BIG = jnp.float32(3.0e38)
  mesh = plsc.VectorSubcoreMesh(core_axis_name="core",
                                subcore_axis_name="subcore",
                                num_cores=NCORE, num_subcores=NSUB)
  cp = pltpu.CompilerParams()
  if "needs_layout_passes" in pltpu.CompilerParams.__dataclass_fields__:
    cp = dataclasses.replace(cp, needs_layout_passes=False)

  @functools.partial(
      pl.kernel,
      out_type=jax.ShapeDtypeStruct((Bn * SURF,), jnp.float32),
      mesh=mesh,
      compiler_params=cp,
      scratch_types=[
          pltpu.VMEM((544,), jnp.int32),
          pltpu.VMEM((E + 16,), jnp.int32),
          pltpu.VMEM((E + 16,), jnp.float32),
          pltpu.VMEM((CPT,), jnp.float32),
          pltpu.VMEM((CPT,), jnp.float32),
          pltpu.VMEM((NLANE,), jnp.float32),
          pltpu.VMEM((NSUB * NLANE,), jnp.float32),
          pltpu.VMEM_SHARED((NSUB * NLANE,), jnp.float32),
          pltpu.SemaphoreType.DMA,
          pltpu.SemaphoreType.DMA,
          pltpu.SemaphoreType.DMA,
      ],
      interpret=interpret,
  )
  def call(si_hbm, st_hbm, bounds_hbm, out_hbm,
           boundsv, cellb, tb, slab, outb, stage, minall, minshared,
           sem_i, sem_t, sem_o):
    c = lax.axis_index("core")
    s = lax.axis_index("subcore")
    pltpu.sync_copy(bounds_hbm, boundsv)
    lanes = lax.iota(jnp.int32, NLANE)

    @pl.loop(0, CPT, step=NLANE, unroll=8)
    def _zero0(i):
      slab[pl.ds(i, NLANE)] = jnp.zeros((NLANE,), jnp.float32)

    def _out_dma(bi):
      dst = out_hbm.at[pl.ds(pl.multiple_of(bi * SURF + s * CPT, 8), CPT)]
      return pltpu.make_async_copy(outb, dst, sem_o)

    @pl.loop(0, BPC)
    def _batch(j):
      bi = c * BPC + j
      k = bi * NSUB + s
      kv = jnp.full((NLANE,), k, jnp.int32)
      bstart = jnp.min(plsc.load_gather(boundsv, [kv]), axis=0)
      bend = jnp.min(plsc.load_gather(boundsv, [kv + 1]), axis=0)
      lo = k * CPT

      start8 = bstart & jnp.int32(-8)
      nch = (bend - start8 + (E - 1)) // E

      def _chunk(k2, macc):
        off = pl.multiple_of(start8 + k2 * E, 8)
        ca = pltpu.make_async_copy(si_hbm.at[pl.ds(off, E + 16)],
                                   cellb, sem_i)
        cb = pltpu.make_async_copy(st_hbm.at[pl.ds(off, E + 16)],
                                   tb, sem_t)
        ca.start()
        cb.start()
        ca.wait()
        cb.wait()

        nv = jnp.minimum(bend - off, E)

        def _vec(v, macc):
          cv = cellb[pl.ds(v, NLANE)]
          tv = tb[pl.ds(v, NLANE)]
          nxt = plsc.load_gather(cellb, [lanes + (v + 1)])
          pos = off + v + lanes
          keep = (pos >= bstart) & (pos < bend) & (cv != nxt)
          loc = jnp.where(keep, cv - lo, 0)
          plsc.store_scatter(slab, [loc], tv, mask=keep)
          return jnp.minimum(macc, jnp.where(keep, tv, BIG))

        return pl.loop(0, nv, step=NLANE, init_carry=macc)(_vec)

      macc = lax.fori_loop(0, nch, _chunk,
                           jnp.full((NLANE,), BIG, jnp.float32))
      lmin = jnp.min(macc, axis=0)
      stage[...] = jnp.full((NLANE,), lmin, jnp.float32)
      pltpu.sync_copy(stage,
                      minshared.at[pl.ds(pl.multiple_of(s * NLANE, 8),
                                         NLANE)])
      plsc.subcore_barrier()
      pltpu.sync_copy(minshared, minall)
      plsc.subcore_barrier()

      def _min2(i, acc):
        return jnp.minimum(acc, minall[pl.ds(i * NLANE, NLANE)])
      acc2 = lax.fori_loop(0, NSUB, _min2,
                           jnp.full((NLANE,), BIG, jnp.float32))
      bmin = jnp.min(acc2, axis=0)

      @pl.when(j > 0)
      def _():
        _out_dma(bi - 1).wait()

      @pl.loop(0, CPT, step=NLANE, unroll=8)
      def _sub(i):
        vv = slab[pl.ds(i, NLANE)]
        outb[pl.ds(i, NLANE)] = jnp.maximum(vv - bmin, jnp.float32(0.0))
        slab[pl.ds(i, NLANE)] = jnp.zeros((NLANE,), jnp.float32)

      _out_dma(bi).start()

    _out_dma(c * BPC + BPC - 1).wait()

  return call


def _run(events, Bn, H, W, BL, E, interpret=False):
  N, five = events.shape
  assert five == 5
  HW = H * W
  SURF = 2 * HW
  CPT = SURF // NSUB
  idx1, t1 = _prep_call(N, BL, W, HW, SURF, interpret=interpret)(events.T)
  si, st = lax.sort([idx1, t1], dimension=0, num_keys=1, is_stable=False)
  nseg = Bn * NSUB
  bounds = jnp.searchsorted(
      si, jnp.arange(nseg + 1, dtype=jnp.int32) * CPT,
      side="left").astype(jnp.int32)
  bounds = jnp.concatenate(
      [bounds, jnp.full((544 - (nseg + 1),), N, jnp.int32)])
  PAD = E + 32
  si_p = jnp.concatenate(
      [si, jnp.full((PAD,), jnp.int32(2**31 - 1))])
  st_p = jnp.concatenate([st, jnp.zeros((PAD,), jnp.float32)])
  out = _sc_call(N + PAD, Bn, SURF, E, interpret=interpret)(
      si_p, st_p, bounds)
  return out.reshape(Bn, 2, H, W)


def kernel(events):
  return _run(events, Bn=32, H=480, W=640, BL=8192, E=8192)


# speedup vs baseline: 5.0222x; 1.0009x over previous
import dataclasses
import functools

import jax
import jax.numpy as jnp
from jax import lax
from jax.experimental import pallas as pl
from jax.experimental.pallas import tpu as pltpu
from jax.experimental.pallas import tpu_sc as plsc

NLANE = 16
NSUB = 16
NCORE = 2


def _prep_call(N, BL, W, HW, SURF, interpret=False):
  G = (N + BL - 1) // BL

  def body(ev_ref, idx_ref, t_ref):
    a = ev_ref[...]
    xi = a[0].astype(jnp.int32)
    yi = a[1].astype(jnp.int32)
    pi = a[3].astype(jnp.int32)
    bi = a[4].astype(jnp.int32)
    idx_ref[...] = bi * SURF + pi * HW + yi * W + xi
    t_ref[...] = a[2]

  return pl.pallas_call(
      body,
      grid=(G,),
      in_specs=[pl.BlockSpec((5, BL), lambda i: (0, i))],
      out_specs=[
          pl.BlockSpec((BL,), lambda i: (i,)),
          pl.BlockSpec((BL,), lambda i: (i,)),
      ],
      out_shape=[
          jax.ShapeDtypeStruct((N,), jnp.int32),
          jax.ShapeDtypeStruct((N,), jnp.float32),
      ],
      interpret=interpret,
  )


def _sc_call(NP, Bn, SURF, E, interpret=False):
  CPT = SURF // NSUB
  BPC = Bn // NCORE
  BIG = jnp.float32(3.0e38)
  mesh = plsc.VectorSubcoreMesh(core_axis_name="core",
                                subcore_axis_name="subcore",
                                num_cores=NCORE, num_subcores=NSUB)
  cp = pltpu.CompilerParams()
  if "needs_layout_passes" in pltpu.CompilerParams.__dataclass_fields__:
    cp = dataclasses.replace(cp, needs_layout_passes=False)

  @functools.partial(
      pl.kernel,
      out_type=jax.ShapeDtypeStruct((Bn * SURF,), jnp.float32),
      mesh=mesh,
      compiler_params=cp,
      scratch_types=[
          pltpu.VMEM((544,), jnp.int32),
          pltpu.VMEM((E + 16,), jnp.int32),
          pltpu.VMEM((E + 16,), jnp.float32),
          pltpu.VMEM((CPT,), jnp.float32),
          pltpu.VMEM((CPT,), jnp.float32),
          pltpu.VMEM((NLANE,), jnp.float32),
          pltpu.VMEM((NSUB * NLANE,), jnp.float32),
          pltpu.VMEM_SHARED((NSUB * NLANE,), jnp.float32),
          pltpu.SemaphoreType.DMA,
          pltpu.SemaphoreType.DMA,
          pltpu.SemaphoreType.DMA,
      ],
      interpret=interpret,
  )
  def call(si_hbm, st_hbm, bounds_hbm, out_hbm,
           boundsv, cellb, tb, slab, outb, stage, minall, minshared,
           sem_i, sem_t, sem_o):
    c = lax.axis_index("core")
    s = lax.axis_index("subcore")
    pltpu.sync_copy(bounds_hbm, boundsv)
    lanes = lax.iota(jnp.int32, NLANE)

    @pl.loop(0, CPT, step=NLANE, unroll=8)
    def _zero0(i):
      slab[pl.ds(i, NLANE)] = jnp.zeros((NLANE,), jnp.float32)

    def _out_dma(bi):
      dst = out_hbm.at[pl.ds(pl.multiple_of(bi * SURF + s * CPT, 8), CPT)]
      return pltpu.make_async_copy(outb, dst, sem_o)

    @pl.loop(0, BPC)
    def _batch(j):
      bi = c * BPC + j
      k = bi * NSUB + s
      kv = jnp.full((NLANE,), k, jnp.int32)
      bstart = jnp.min(plsc.load_gather(boundsv, [kv]), axis=0)
      bend = jnp.min(plsc.load_gather(boundsv, [kv + 1]), axis=0)
      lo = k * CPT

      start8 = bstart & jnp.int32(-8)
      nch = (bend - start8 + (E - 1)) // E

      def _chunk(k2, macc):
        off = pl.multiple_of(start8 + k2 * E, 8)
        ca = pltpu.make_async_copy(si_hbm.at[pl.ds(off, E + 16)],
                                   cellb, sem_i)
        cb = pltpu.make_async_copy(st_hbm.at[pl.ds(off, E + 16)],
                                   tb, sem_t)
        ca.start()
        cb.start()
        ca.wait()
        cb.wait()

        nv = jnp.minimum(bend - off, E)

        def _vec(v, macc):
          cv = cellb[pl.ds(v, NLANE)]
          tv = tb[pl.ds(v, NLANE)]
          nxt = plsc.load_gather(cellb, [lanes + (v + 1)])
          pos = off + v + lanes
          keep = (pos >= bstart) & (pos < bend) & (cv != nxt)
          loc = jnp.where(keep, cv - lo, 0)
          plsc.store_scatter(slab, [loc], tv, mask=keep)
          return jnp.minimum(macc, jnp.where(keep, tv, BIG))

        return pl.loop(0, nv, step=NLANE, init_carry=macc)(_vec)

      macc = lax.fori_loop(0, nch, _chunk,
                           jnp.full((NLANE,), BIG, jnp.float32))
      lmin = jnp.min(macc, axis=0)
      stage[...] = jnp.full((NLANE,), lmin, jnp.float32)
      pltpu.sync_copy(stage,
                      minshared.at[pl.ds(pl.multiple_of(s * NLANE, 8),
                                         NLANE)])
      plsc.subcore_barrier()
      pltpu.sync_copy(minshared, minall)
      plsc.subcore_barrier()

      def _min2(i, acc):
        return jnp.minimum(acc, minall[pl.ds(i * NLANE, NLANE)])
      acc2 = lax.fori_loop(0, NSUB, _min2,
                           jnp.full((NLANE,), BIG, jnp.float32))
      bmin = jnp.min(acc2, axis=0)

      @pl.when(j > 0)
      def _():
        _out_dma(bi - 1).wait()

      @pl.loop(0, CPT, step=NLANE, unroll=8)
      def _sub(i):
        vv = slab[pl.ds(i, NLANE)]
        outb[pl.ds(i, NLANE)] = jnp.maximum(vv - bmin, jnp.float32(0.0))
        slab[pl.ds(i, NLANE)] = jnp.zeros((NLANE,), jnp.float32)

      _out_dma(bi).start()

    _out_dma(c * BPC + BPC - 1).wait()

  return call


def _run(events, Bn, H, W, BL, E, interpret=False):
  N, five = events.shape
  assert five == 5
  HW = H * W
  SURF = 2 * HW
  CPT = SURF // NSUB
  idx1, t1 = _prep_call(N, BL, W, HW, SURF, interpret=interpret)(events.T)
  si, st = lax.sort([idx1, t1], dimension=0, num_keys=1, is_stable=False)
  nseg = Bn * NSUB
  bounds = jnp.searchsorted(
      si, jnp.arange(nseg + 1, dtype=jnp.int32) * CPT,
      side="left").astype(jnp.int32)
  bounds = jnp.concatenate(
      [bounds, jnp.full((544 - (nseg + 1),), N, jnp.int32)])
  PAD = E + 32
  si_p = jnp.concatenate(
      [si, jnp.full((PAD,), jnp.int32(2**31 - 1))])
  st_p = jnp.concatenate([st, jnp.zeros((PAD,), jnp.float32)])
  out = _sc_call(N + PAD, Bn, SURF, E, interpret=interpret)(
      si_p, st_p, bounds)
  return out.reshape(Bn, 2, H, W)


def kernel(events):
  return _run(events, Bn=32, H=480, W=640, BL=8192, E=8192)
